# baseline (speedup 1.0000x reference)
import functools

import jax
import jax.numpy as jnp
from jax import lax
from jax.experimental import pallas as pl
from jax.experimental.pallas import tpu as pltpu
from jax.experimental.pallas import tpu_sc as plsc

NC = 2
NS = 16
NW = NC * NS


def _mesh():
    return plsc.VectorSubcoreMesh(core_axis_name="c", subcore_axis_name="s")


def _sc_degree(epk, ones, zeros):
    b = epk.shape[2]
    nch = epk.shape[0] // NW
    npad, h = zeros.shape
    rows_per_tile = npad // NS

    @functools.partial(
        pl.kernel,
        out_type=jax.ShapeDtypeStruct((NC, npad, h), jnp.float32),
        mesh=_mesh(),
        scratch_types=[
            pltpu.VMEM((b,), jnp.int32),
            pltpu.VMEM((b, h), jnp.float32),
            pltpu.VMEM_SHARED((npad, h), jnp.float32),
        ],
    )
    def k(epk_hbm, ones_hbm, zeros_hbm, out_hbm, didx, ones_v, acc):
        cid = lax.axis_index("c")
        sid = lax.axis_index("s")
        rbase = sid * rows_per_tile

        pltpu.sync_copy(ones_hbm, ones_v)
        pltpu.sync_copy(zeros_hbm.at[pl.ds(rbase, rows_per_tile)],
                        acc.at[pl.ds(rbase, rows_per_tile)])
        plsc.subcore_barrier()

        cbase = (cid * NS + sid) * nch

        def chunk_body(ci, c):
            pltpu.sync_copy(epk_hbm.at[cbase + ci, 1], didx)
            pltpu.sync_copy(ones_v, acc.at[didx], add=True)
            return c

        lax.fori_loop(0, nch, chunk_body, 0)
        plsc.subcore_barrier()
        pltpu.sync_copy(acc.at[pl.ds(rbase, rows_per_tile)],
                        out_hbm.at[cid, pl.ds(rbase, rows_per_tile)])

    return k(epk, ones, zeros)


def _sc_propagate(mp, epk):
    npad, h = mp.shape
    b = epk.shape[2]
    nch = epk.shape[0] // NW
    rows_per_tile = npad // NS

    @functools.partial(
        pl.kernel,
        out_type=jax.ShapeDtypeStruct((NC, npad, h), jnp.float32),
        mesh=_mesh(),
        scratch_types=[
            pltpu.VMEM((b,), jnp.int32),
            pltpu.VMEM((b,), jnp.int32),
            pltpu.VMEM((b,), jnp.int32),
            pltpu.VMEM((b,), jnp.int32),
            pltpu.VMEM((b, h), jnp.float32),
            pltpu.VMEM((b, h), jnp.float32),
            pltpu.VMEM_SHARED((npad, h), jnp.float32),
            pltpu.SemaphoreType.DMA,
            pltpu.SemaphoreType.DMA,
        ],
    )
    def k(mp_hbm, epk_hbm, out_hbm, sidx0, sidx1, didx0, didx1,
          rows0, rows1, acc, g0, g1):
        cid = lax.axis_index("c")
        sid = lax.axis_index("s")
        rbase = sid * rows_per_tile

        pltpu.sync_copy(mp_hbm.at[pl.ds(rbase, rows_per_tile)],
                        acc.at[pl.ds(rbase, rows_per_tile)])
        plsc.subcore_barrier()

        cbase = (cid * NS + sid) * nch

        def stage(ci, sidx, didx):
            pltpu.sync_copy(epk_hbm.at[ci, 0], sidx)
            pltpu.sync_copy(epk_hbm.at[ci, 1], didx)

        def gather(sidx, rows, sem):
            return pltpu.async_copy(mp_hbm.at[sidx], rows, sem)

        def gwait(sidx, rows, sem):
            pltpu.make_async_copy(mp_hbm.at[sidx], rows, sem).wait()

        def scatter(rows, didx):
            pltpu.sync_copy(rows, acc.at[didx], add=True)

        stage(cbase, sidx0, didx0)
        gather(sidx0, rows0, g0)

        def pair_body(j, c):
            c0 = cbase + 2 * j
            stage(c0 + 1, sidx1, didx1)
            gwait(sidx0, rows0, g0)
            gather(sidx1, rows1, g1)
            scatter(rows0, didx0)
            stage(c0 + 2, sidx0, didx0)
            gwait(sidx1, rows1, g1)
            gather(sidx0, rows0, g0)
            scatter(rows1, didx1)
            return c

        lax.fori_loop(0, nch // 2 - 1, pair_body, 0)

        stage(cbase + nch - 1, sidx1, didx1)
        gwait(sidx0, rows0, g0)
        gather(sidx1, rows1, g1)
        scatter(rows0, didx0)
        gwait(sidx1, rows1, g1)
        scatter(rows1, didx1)

        plsc.subcore_barrier()
        pltpu.sync_copy(acc.at[pl.ds(rbase, rows_per_tile)],
                        out_hbm.at[cid, pl.ds(rbase, rows_per_tile)])

    return k(mp, epk)


def _tc_dis(degp):
    _, n, _ = degp.shape

    def body(degp_ref, dis_ref):
        deg = degp_ref[0, :, 0:1] + degp_ref[1, :, 0:1] + 1.0
        dis_ref[...] = lax.rsqrt(deg)

    return pl.pallas_call(
        body,
        out_shape=jax.ShapeDtypeStruct((n, 1), jnp.float32),
    )(degp)


def _tc_prep(x, w1, dis, blk):
    n, d = x.shape
    h = w1.shape[1]
    grid = n // blk

    def body(x_ref, w_ref, dis_ref, mp_ref):
        mp_ref[...] = dis_ref[...] * jnp.dot(x_ref[...], w_ref[...],
                                             preferred_element_type=jnp.float32)

    return pl.pallas_call(
        body,
        grid=(grid,),
        in_specs=[
            pl.BlockSpec((blk, d), lambda i: (i, 0)),
            pl.BlockSpec((d, h), lambda i: (0, 0)),
            pl.BlockSpec((blk, 1), lambda i: (i, 0)),
        ],
        out_specs=pl.BlockSpec((blk, h), lambda i: (i, 0)),
        out_shape=jax.ShapeDtypeStruct((n, h), jnp.float32),
    )(x, w1, dis)


def _tc_layer(s, mp, dis, b_prev, w_next, blk):
    _, n, h = s.shape
    grid = n // blk

    def body(s_ref, mp_ref, dis_ref, b_ref, w_ref, out_ref):
        dis = dis_ref[...]
        hpre = dis * (s_ref[0] + s_ref[1] - mp_ref[...]) + b_ref[...]
        act = jnp.maximum(hpre, 0.0)
        out_ref[...] = dis * jnp.dot(act, w_ref[...],
                                     preferred_element_type=jnp.float32)

    return pl.pallas_call(
        body,
        grid=(grid,),
        in_specs=[
            pl.BlockSpec((NC, blk, h), lambda i: (0, i, 0)),
            pl.BlockSpec((blk, h), lambda i: (i, 0)),
            pl.BlockSpec((blk, 1), lambda i: (i, 0)),
            pl.BlockSpec((1, h), lambda i: (0, 0)),
            pl.BlockSpec((h, w_next.shape[1]), lambda i: (0, 0)),
        ],
        out_specs=pl.BlockSpec((blk, h), lambda i: (i, 0)),
        out_shape=jax.ShapeDtypeStruct((n, h), jnp.float32),
    )(s, mp, dis, b_prev, w_next)


def _tc_final(s, mp, dis, b6, batch3d, wlin, blin, n_graphs, blk):
    _, n, h = s.shape
    c = wlin.shape[1]
    grid = n // blk

    def body(s_ref, mp_ref, dis_ref, b_ref, batch_ref, wlin_ref, blin_ref,
             out_ref, pool_acc, cnt_acc):
        i = pl.program_id(0)

        @pl.when(i == 0)
        def _():
            pool_acc[...] = jnp.zeros_like(pool_acc)
            cnt_acc[...] = jnp.zeros_like(cnt_acc)

        h6 = dis_ref[...] * (s_ref[0] + s_ref[1] - mp_ref[...]) + b_ref[...]
        gids = lax.broadcasted_iota(jnp.int32, (n_graphs, blk), 0)
        onehot = (batch_ref[0] == gids).astype(jnp.float32)
        pool_acc[...] += jnp.dot(onehot, h6, preferred_element_type=jnp.float32)
        cnt_acc[...] += jnp.sum(onehot, axis=1, keepdims=True)

        @pl.when(i == grid - 1)
        def _():
            pooled = pool_acc[...] / jnp.maximum(cnt_acc[...], 1.0)
            out_ref[...] = jnp.dot(pooled, wlin_ref[...],
                                   preferred_element_type=jnp.float32) + blin_ref[...]

    return pl.pallas_call(
        body,
        grid=(grid,),
        in_specs=[
            pl.BlockSpec((NC, blk, h), lambda i: (0, i, 0)),
            pl.BlockSpec((blk, h), lambda i: (i, 0)),
            pl.BlockSpec((blk, 1), lambda i: (i, 0)),
            pl.BlockSpec((1, h), lambda i: (0, 0)),
            pl.BlockSpec((1, 1, blk), lambda i: (i, 0, 0)),
            pl.BlockSpec((h, c), lambda i: (0, 0)),
            pl.BlockSpec((1, c), lambda i: (0, 0)),
        ],
        out_specs=pl.BlockSpec((n_graphs, c), lambda i: (0, 0)),
        out_shape=jax.ShapeDtypeStruct((n_graphs, c), jnp.float32),
        scratch_shapes=[
            pltpu.VMEM((n_graphs, h), jnp.float32),
            pltpu.VMEM((n_graphs, 1), jnp.float32),
        ],
    )(s, mp, dis, b6, batch3d, wlin, blin)


def kernel(x, edge_index, batch, W1, b1, W2, b2, W3, b3, W4, b4, W5, b5,
           W6, b6, Wlin, blin):
    n, d = x.shape
    g = 64
    h = W1.shape[1]
    npad = 10240
    blk = 1024
    src = edge_index[0]
    dst = edge_index[1]

    x_p = jnp.pad(x, ((0, npad - n), (0, 0)))
    batch_p = jnp.pad(batch, (0, npad - n), constant_values=g)
    batch3d = batch_p.reshape(npad // blk, 1, blk)
    ones = jnp.ones((128, h), jnp.float32)
    zeros = jnp.zeros((npad, h), jnp.float32)

    e = src.shape[0]
    bch = 128
    ept = (e + NW - 1) // NW
    nch_t = ((ept + bch - 1) // bch + 1) // 2 * 2
    e_pad = NW * nch_t * bch
    src_p = jnp.pad(src, (0, e_pad - e), constant_values=npad - 1)
    dst_p = jnp.pad(dst, (0, e_pad - e), constant_values=npad - 1)
    epk = jnp.stack([src_p.reshape(-1, bch), dst_p.reshape(-1, bch)], axis=1)

    degp = _sc_degree(epk, ones, zeros)
    dis = _tc_dis(degp)
    mp = _tc_prep(x_p, W1, dis, blk)

    for b_prev, w_next in ((b1, W2), (b2, W3), (b3, W4), (b4, W5), (b5, W6)):
        s = _sc_propagate(mp, epk)
        mp = _tc_layer(s, mp, dis, b_prev.reshape(1, -1), w_next, blk)

    s = _sc_propagate(mp, epk)
    return _tc_final(s, mp, dis, b6.reshape(1, -1), batch3d, Wlin,
                     blin.reshape(1, -1), g, blk)


# speedup vs baseline: 2.6583x; 2.6583x over previous
import functools

import jax
import jax.numpy as jnp
from jax import lax
from jax.experimental import pallas as pl
from jax.experimental.pallas import tpu as pltpu
from jax.experimental.pallas import tpu_sc as plsc

NC = 2
NS = 16
NW = NC * NS


def _mesh():
    return plsc.VectorSubcoreMesh(core_axis_name="c", subcore_axis_name="s")


def _sc_degree(epk, ones, zeros):
    b = epk.shape[2]
    nch = epk.shape[0] // NW
    npad, h = zeros.shape
    rows_per_tile = npad // NS

    @functools.partial(
        pl.kernel,
        out_type=jax.ShapeDtypeStruct((NC, npad, h), jnp.float32),
        mesh=_mesh(),
        scratch_types=[
            pltpu.VMEM((b,), jnp.int32),
            pltpu.VMEM((b, h), jnp.float32),
            pltpu.VMEM_SHARED((npad, h), jnp.float32),
        ],
    )
    def k(epk_hbm, ones_hbm, zeros_hbm, out_hbm, didx, ones_v, acc):
        cid = lax.axis_index("c")
        sid = lax.axis_index("s")
        rbase = sid * rows_per_tile

        pltpu.sync_copy(ones_hbm, ones_v)
        pltpu.sync_copy(zeros_hbm.at[pl.ds(rbase, rows_per_tile)],
                        acc.at[pl.ds(rbase, rows_per_tile)])
        plsc.subcore_barrier()

        cbase = (cid * NS + sid) * nch

        def chunk_body(ci, c):
            pltpu.sync_copy(epk_hbm.at[cbase + ci, 1], didx)
            pltpu.sync_copy(ones_v, acc.at[didx], add=True)
            return c

        lax.fori_loop(0, nch, chunk_body, 0)
        plsc.subcore_barrier()
        pltpu.sync_copy(acc.at[pl.ds(rbase, rows_per_tile)],
                        out_hbm.at[cid, pl.ds(rbase, rows_per_tile)])

    return k(epk, ones, zeros)


def _sc_propagate(mp, epk):
    npad, h = mp.shape
    b = epk.shape[2]
    nch = epk.shape[0] // NW
    rows_per_tile = npad // NS

    @functools.partial(
        pl.kernel,
        out_type=jax.ShapeDtypeStruct((NC, npad, h), jnp.float32),
        mesh=_mesh(),
        scratch_types=[
            pltpu.VMEM((b,), jnp.int32),
            pltpu.VMEM((b,), jnp.int32),
            pltpu.VMEM((b,), jnp.int32),
            pltpu.VMEM((b,), jnp.int32),
            pltpu.VMEM((b, h), jnp.float32),
            pltpu.VMEM((b, h), jnp.float32),
            pltpu.VMEM_SHARED((npad, h), jnp.float32),
            pltpu.SemaphoreType.DMA,
            pltpu.SemaphoreType.DMA,
        ],
    )
    def k(mp_hbm, epk_hbm, out_hbm, sidx0, sidx1, didx0, didx1,
          rows0, rows1, acc, g0, g1):
        cid = lax.axis_index("c")
        sid = lax.axis_index("s")
        rbase = sid * rows_per_tile

        pltpu.sync_copy(mp_hbm.at[pl.ds(rbase, rows_per_tile)],
                        acc.at[pl.ds(rbase, rows_per_tile)])
        plsc.subcore_barrier()

        cbase = (cid * NS + sid) * nch

        def stage(ci, sidx, didx):
            pltpu.sync_copy(epk_hbm.at[ci, 0], sidx)
            pltpu.sync_copy(epk_hbm.at[ci, 1], didx)

        def gather(sidx, rows, sem):
            return pltpu.async_copy(mp_hbm.at[sidx], rows, sem)

        def gwait(sidx, rows, sem):
            pltpu.make_async_copy(mp_hbm.at[sidx], rows, sem).wait()

        def scatter(rows, didx):
            pltpu.sync_copy(rows, acc.at[didx], add=True)

        stage(cbase, sidx0, didx0)
        gather(sidx0, rows0, g0)

        def pair_body(j, c):
            c0 = cbase + 2 * j
            stage(c0 + 1, sidx1, didx1)
            gwait(sidx0, rows0, g0)
            gather(sidx1, rows1, g1)
            scatter(rows0, didx0)
            stage(c0 + 2, sidx0, didx0)
            gwait(sidx1, rows1, g1)
            gather(sidx0, rows0, g0)
            scatter(rows1, didx1)
            return c

        lax.fori_loop(0, nch // 2 - 1, pair_body, 0)

        stage(cbase + nch - 1, sidx1, didx1)
        gwait(sidx0, rows0, g0)
        gather(sidx1, rows1, g1)
        scatter(rows0, didx0)
        gwait(sidx1, rows1, g1)
        scatter(rows1, didx1)

        plsc.subcore_barrier()
        pltpu.sync_copy(acc.at[pl.ds(rbase, rows_per_tile)],
                        out_hbm.at[cid, pl.ds(rbase, rows_per_tile)])

    return k(mp, epk)


def _tc_dis(degp):
    _, n, _ = degp.shape

    def body(degp_ref, dis_ref):
        deg = degp_ref[0, :, 0:1] + degp_ref[1, :, 0:1] + 1.0
        dis_ref[...] = lax.rsqrt(deg)

    return pl.pallas_call(
        body,
        out_shape=jax.ShapeDtypeStruct((n, 1), jnp.float32),
    )(degp)


def _tc_prep(x, w1, dis, blk):
    n, d = x.shape
    h = w1.shape[1]
    grid = n // blk

    def body(x_ref, w_ref, dis_ref, mp_ref):
        mp_ref[...] = dis_ref[...] * jnp.dot(x_ref[...], w_ref[...],
                                             preferred_element_type=jnp.float32)

    return pl.pallas_call(
        body,
        grid=(grid,),
        in_specs=[
            pl.BlockSpec((blk, d), lambda i: (i, 0)),
            pl.BlockSpec((d, h), lambda i: (0, 0)),
            pl.BlockSpec((blk, 1), lambda i: (i, 0)),
        ],
        out_specs=pl.BlockSpec((blk, h), lambda i: (i, 0)),
        out_shape=jax.ShapeDtypeStruct((n, h), jnp.float32),
    )(x, w1, dis)


def _tc_layer(s, mp, dis, b_prev, w_next, blk):
    _, n, h = s.shape
    grid = n // blk

    def body(s_ref, mp_ref, dis_ref, b_ref, w_ref, out_ref):
        dis = dis_ref[...]
        hpre = dis * (s_ref[0] + s_ref[1] - mp_ref[...]) + b_ref[...]
        act = jnp.maximum(hpre, 0.0)
        out_ref[...] = dis * jnp.dot(act, w_ref[...],
                                     preferred_element_type=jnp.float32)

    return pl.pallas_call(
        body,
        grid=(grid,),
        in_specs=[
            pl.BlockSpec((NC, blk, h), lambda i: (0, i, 0)),
            pl.BlockSpec((blk, h), lambda i: (i, 0)),
            pl.BlockSpec((blk, 1), lambda i: (i, 0)),
            pl.BlockSpec((1, h), lambda i: (0, 0)),
            pl.BlockSpec((h, w_next.shape[1]), lambda i: (0, 0)),
        ],
        out_specs=pl.BlockSpec((blk, h), lambda i: (i, 0)),
        out_shape=jax.ShapeDtypeStruct((n, h), jnp.float32),
    )(s, mp, dis, b_prev, w_next)


def _tc_final(s, mp, dis, b6, batch3d, wlin, blin, n_graphs, blk):
    _, n, h = s.shape
    c = wlin.shape[1]
    grid = n // blk

    def body(s_ref, mp_ref, dis_ref, b_ref, batch_ref, wlin_ref, blin_ref,
             out_ref, pool_acc, cnt_acc):
        i = pl.program_id(0)

        @pl.when(i == 0)
        def _():
            pool_acc[...] = jnp.zeros_like(pool_acc)
            cnt_acc[...] = jnp.zeros_like(cnt_acc)

        h6 = dis_ref[...] * (s_ref[0] + s_ref[1] - mp_ref[...]) + b_ref[...]
        gids = lax.broadcasted_iota(jnp.int32, (n_graphs, blk), 0)
        onehot = (batch_ref[0] == gids).astype(jnp.float32)
        pool_acc[...] += jnp.dot(onehot, h6, preferred_element_type=jnp.float32)
        cnt_acc[...] += jnp.sum(onehot, axis=1, keepdims=True)

        @pl.when(i == grid - 1)
        def _():
            pooled = pool_acc[...] / jnp.maximum(cnt_acc[...], 1.0)
            out_ref[...] = jnp.dot(pooled, wlin_ref[...],
                                   preferred_element_type=jnp.float32) + blin_ref[...]

    return pl.pallas_call(
        body,
        grid=(grid,),
        in_specs=[
            pl.BlockSpec((NC, blk, h), lambda i: (0, i, 0)),
            pl.BlockSpec((blk, h), lambda i: (i, 0)),
            pl.BlockSpec((blk, 1), lambda i: (i, 0)),
            pl.BlockSpec((1, h), lambda i: (0, 0)),
            pl.BlockSpec((1, 1, blk), lambda i: (i, 0, 0)),
            pl.BlockSpec((h, c), lambda i: (0, 0)),
            pl.BlockSpec((1, c), lambda i: (0, 0)),
        ],
        out_specs=pl.BlockSpec((n_graphs, c), lambda i: (0, 0)),
        out_shape=jax.ShapeDtypeStruct((n_graphs, c), jnp.float32),
        scratch_shapes=[
            pltpu.VMEM((n_graphs, h), jnp.float32),
            pltpu.VMEM((n_graphs, 1), jnp.float32),
        ],
    )(s, mp, dis, b6, batch3d, wlin, blin)


def kernel(x, edge_index, batch, W1, b1, W2, b2, W3, b3, W4, b4, W5, b5,
           W6, b6, Wlin, blin):
    n, d = x.shape
    g = 64
    h = W1.shape[1]
    npad = 10240
    blk = 1024
    src = edge_index[0]
    dst = edge_index[1]

    x_p = jnp.pad(x, ((0, npad - n), (0, 0)))
    batch_p = jnp.pad(batch, (0, npad - n), constant_values=g)
    batch3d = batch_p.reshape(npad // blk, 1, blk)
    ones = jnp.ones((128, h), jnp.float32)
    zeros = jnp.zeros((npad, h), jnp.float32)

    e = src.shape[0]
    bch = 128
    ept = e // NW
    nch_t = ((ept + bch - 1) // bch + 1) // 2 * 2
    pad_t = nch_t * bch - ept
    pad_ids = n + (jnp.arange(pad_t, dtype=jnp.int32) % (npad - n))
    pad_blk = jnp.broadcast_to(pad_ids, (NW, pad_t))
    src_t = jnp.concatenate([src.reshape(NW, ept), pad_blk], axis=1)
    dst_t = jnp.concatenate([dst.reshape(NW, ept), pad_blk], axis=1)
    epk = jnp.stack([src_t.reshape(-1, bch), dst_t.reshape(-1, bch)], axis=1)

    degp = _sc_degree(epk, ones, zeros)
    dis = _tc_dis(degp)
    mp = _tc_prep(x_p, W1, dis, blk)

    for b_prev, w_next in ((b1, W2), (b2, W3), (b3, W4), (b4, W5), (b5, W6)):
        s = _sc_propagate(mp, epk)
        mp = _tc_layer(s, mp, dis, b_prev.reshape(1, -1), w_next, blk)

    s = _sc_propagate(mp, epk)
    return _tc_final(s, mp, dis, b6.reshape(1, -1), batch3d, Wlin,
                     blin.reshape(1, -1), g, blk)


# speedup vs baseline: 2.7583x; 1.0376x over previous
import functools

import jax
import jax.numpy as jnp
from jax import lax
from jax.experimental import pallas as pl
from jax.experimental.pallas import tpu as pltpu
from jax.experimental.pallas import tpu_sc as plsc

NC = 2
NS = 16
NW = NC * NS


def _mesh():
    return plsc.VectorSubcoreMesh(core_axis_name="c", subcore_axis_name="s")


def _sc_degree(epk, ones, zeros):
    b = epk.shape[2]
    nch = epk.shape[0] // NW
    npad, h = zeros.shape
    rows_per_tile = npad // NS

    @functools.partial(
        pl.kernel,
        out_type=jax.ShapeDtypeStruct((NC, npad, h), jnp.float32),
        mesh=_mesh(),
        scratch_types=[
            pltpu.VMEM((b,), jnp.int32),
            pltpu.VMEM((b, h), jnp.float32),
            pltpu.VMEM_SHARED((npad, h), jnp.float32),
        ],
    )
    def k(epk_hbm, ones_hbm, zeros_hbm, out_hbm, didx, ones_v, acc):
        cid = lax.axis_index("c")
        sid = lax.axis_index("s")
        rbase = sid * rows_per_tile

        pltpu.sync_copy(ones_hbm, ones_v)
        pltpu.sync_copy(zeros_hbm.at[pl.ds(rbase, rows_per_tile)],
                        acc.at[pl.ds(rbase, rows_per_tile)])
        plsc.subcore_barrier()

        cbase = (cid * NS + sid) * nch

        def chunk_body(ci, c):
            pltpu.sync_copy(epk_hbm.at[cbase + ci, 1], didx)
            pltpu.sync_copy(ones_v, acc.at[didx], add=True)
            return c

        lax.fori_loop(0, nch, chunk_body, 0)
        plsc.subcore_barrier()
        pltpu.sync_copy(acc.at[pl.ds(rbase, rows_per_tile)],
                        out_hbm.at[cid, pl.ds(rbase, rows_per_tile)])

    return k(epk, ones, zeros)


def _sc_propagate(mp, epk):
    npad, h = mp.shape
    b = epk.shape[2]
    nch = epk.shape[0] // NW
    rows_per_tile = npad // NS

    @functools.partial(
        pl.kernel,
        out_type=jax.ShapeDtypeStruct((NC, npad, h), jnp.float32),
        mesh=_mesh(),
        scratch_types=[
            pltpu.VMEM((b,), jnp.int32),
            pltpu.VMEM((b,), jnp.int32),
            pltpu.VMEM((b,), jnp.int32),
            pltpu.VMEM((b,), jnp.int32),
            pltpu.VMEM((b, h), jnp.float32),
            pltpu.VMEM((b, h), jnp.float32),
            pltpu.VMEM_SHARED((npad, h), jnp.float32),
            pltpu.SemaphoreType.DMA,
            pltpu.SemaphoreType.DMA,
        ],
    )
    def k(mp_hbm, epk_hbm, out_hbm, sidx0, sidx1, didx0, didx1,
          rows0, rows1, acc, g0, g1):
        cid = lax.axis_index("c")
        sid = lax.axis_index("s")
        rbase = sid * rows_per_tile

        pltpu.sync_copy(mp_hbm.at[pl.ds(rbase, rows_per_tile)],
                        acc.at[pl.ds(rbase, rows_per_tile)])
        plsc.subcore_barrier()

        cbase = (cid * NS + sid) * nch

        def stage(ci, sidx, didx):
            pltpu.sync_copy(epk_hbm.at[ci, 0], sidx)
            pltpu.sync_copy(epk_hbm.at[ci, 1], didx)

        def gather(sidx, rows, sem):
            return pltpu.async_copy(mp_hbm.at[sidx], rows, sem)

        def gwait(sidx, rows, sem):
            pltpu.make_async_copy(mp_hbm.at[sidx], rows, sem).wait()

        def scatter(rows, didx):
            pltpu.sync_copy(rows, acc.at[didx], add=True)

        stage(cbase, sidx0, didx0)
        gather(sidx0, rows0, g0)

        def pair_body(j, c):
            c0 = cbase + 2 * j
            stage(c0 + 1, sidx1, didx1)
            gwait(sidx0, rows0, g0)
            gather(sidx1, rows1, g1)
            scatter(rows0, didx0)
            stage(c0 + 2, sidx0, didx0)
            gwait(sidx1, rows1, g1)
            gather(sidx0, rows0, g0)
            scatter(rows1, didx1)
            return c

        lax.fori_loop(0, nch // 2 - 1, pair_body, 0)

        stage(cbase + nch - 1, sidx1, didx1)
        gwait(sidx0, rows0, g0)
        gather(sidx1, rows1, g1)
        scatter(rows0, didx0)
        gwait(sidx1, rows1, g1)
        scatter(rows1, didx1)

        plsc.subcore_barrier()
        pltpu.sync_copy(acc.at[pl.ds(rbase, rows_per_tile)],
                        out_hbm.at[cid, pl.ds(rbase, rows_per_tile)])

    return k(mp, epk)


def _tc_dis(degp):
    _, n, _ = degp.shape

    def body(degp_ref, dis_ref):
        deg = degp_ref[0, :, 0:1] + degp_ref[1, :, 0:1] + 1.0
        dis_ref[...] = lax.rsqrt(deg)

    return pl.pallas_call(
        body,
        out_shape=jax.ShapeDtypeStruct((n, 1), jnp.float32),
    )(degp)


def _tc_prep(x, w1, dis, blk):
    n, d = x.shape
    h = w1.shape[1]
    grid = n // blk

    def body(x_ref, w_ref, dis_ref, mp_ref):
        mp_ref[...] = dis_ref[...] * jnp.dot(x_ref[...], w_ref[...],
                                             preferred_element_type=jnp.float32)

    return pl.pallas_call(
        body,
        grid=(grid,),
        in_specs=[
            pl.BlockSpec((blk, d), lambda i: (i, 0)),
            pl.BlockSpec((d, h), lambda i: (0, 0)),
            pl.BlockSpec((blk, 1), lambda i: (i, 0)),
        ],
        out_specs=pl.BlockSpec((blk, h), lambda i: (i, 0)),
        out_shape=jax.ShapeDtypeStruct((n, h), jnp.float32),
    )(x, w1, dis)


def _tc_layer(s, mp, dis, b_prev, w_next, blk):
    _, n, h = s.shape
    grid = n // blk

    def body(s_ref, mp_ref, dis_ref, b_ref, w_ref, out_ref):
        dis = dis_ref[...]
        hpre = dis * (s_ref[0] + s_ref[1] - mp_ref[...]) + b_ref[...]
        act = jnp.maximum(hpre, 0.0)
        out_ref[...] = dis * jnp.dot(act, w_ref[...],
                                     preferred_element_type=jnp.float32)

    return pl.pallas_call(
        body,
        grid=(grid,),
        in_specs=[
            pl.BlockSpec((NC, blk, h), lambda i: (0, i, 0)),
            pl.BlockSpec((blk, h), lambda i: (i, 0)),
            pl.BlockSpec((blk, 1), lambda i: (i, 0)),
            pl.BlockSpec((1, h), lambda i: (0, 0)),
            pl.BlockSpec((h, w_next.shape[1]), lambda i: (0, 0)),
        ],
        out_specs=pl.BlockSpec((blk, h), lambda i: (i, 0)),
        out_shape=jax.ShapeDtypeStruct((n, h), jnp.float32),
    )(s, mp, dis, b_prev, w_next)


def _tc_final(s, mp, dis, b6, batch3d, wlin, blin, n_graphs, blk):
    _, n, h = s.shape
    c = wlin.shape[1]
    grid = n // blk

    def body(s_ref, mp_ref, dis_ref, b_ref, batch_ref, wlin_ref, blin_ref,
             out_ref, pool_acc, cnt_acc):
        i = pl.program_id(0)

        @pl.when(i == 0)
        def _():
            pool_acc[...] = jnp.zeros_like(pool_acc)
            cnt_acc[...] = jnp.zeros_like(cnt_acc)

        h6 = dis_ref[...] * (s_ref[0] + s_ref[1] - mp_ref[...]) + b_ref[...]
        gids = lax.broadcasted_iota(jnp.int32, (n_graphs, blk), 0)
        onehot = (batch_ref[0] == gids).astype(jnp.float32)
        pool_acc[...] += jnp.dot(onehot, h6, preferred_element_type=jnp.float32)
        cnt_acc[...] += jnp.sum(onehot, axis=1, keepdims=True)

        @pl.when(i == grid - 1)
        def _():
            pooled = pool_acc[...] / jnp.maximum(cnt_acc[...], 1.0)
            out_ref[...] = jnp.dot(pooled, wlin_ref[...],
                                   preferred_element_type=jnp.float32) + blin_ref[...]

    return pl.pallas_call(
        body,
        grid=(grid,),
        in_specs=[
            pl.BlockSpec((NC, blk, h), lambda i: (0, i, 0)),
            pl.BlockSpec((blk, h), lambda i: (i, 0)),
            pl.BlockSpec((blk, 1), lambda i: (i, 0)),
            pl.BlockSpec((1, h), lambda i: (0, 0)),
            pl.BlockSpec((1, 1, blk), lambda i: (i, 0, 0)),
            pl.BlockSpec((h, c), lambda i: (0, 0)),
            pl.BlockSpec((1, c), lambda i: (0, 0)),
        ],
        out_specs=pl.BlockSpec((n_graphs, c), lambda i: (0, 0)),
        out_shape=jax.ShapeDtypeStruct((n_graphs, c), jnp.float32),
        scratch_shapes=[
            pltpu.VMEM((n_graphs, h), jnp.float32),
            pltpu.VMEM((n_graphs, 1), jnp.float32),
        ],
    )(s, mp, dis, b6, batch3d, wlin, blin)


def kernel(x, edge_index, batch, W1, b1, W2, b2, W3, b3, W4, b4, W5, b5,
           W6, b6, Wlin, blin):
    n, d = x.shape
    g = 64
    h = W1.shape[1]
    npad = 10240
    blk = 1024
    src = edge_index[0]
    dst = edge_index[1]

    x_p = jnp.pad(x, ((0, npad - n), (0, 0)))
    batch_p = jnp.pad(batch, (0, npad - n), constant_values=g)
    batch3d = batch_p.reshape(npad // blk, 1, blk)
    ones = jnp.ones((128, 16), jnp.float32)
    zeros = jnp.zeros((npad, 16), jnp.float32)

    e = src.shape[0]
    bch = 128
    ept = e // NW
    nch_t = ((ept + bch - 1) // bch + 1) // 2 * 2
    pad_t = nch_t * bch - ept
    pad_ids = n + (jnp.arange(pad_t, dtype=jnp.int32) % (npad - n))
    pad_blk = jnp.broadcast_to(pad_ids, (NW, pad_t))
    src_t = jnp.concatenate([src.reshape(NW, ept), pad_blk], axis=1)
    dst_t = jnp.concatenate([dst.reshape(NW, ept), pad_blk], axis=1)
    epk = jnp.stack([src_t.reshape(-1, bch), dst_t.reshape(-1, bch)], axis=1)

    degp = _sc_degree(epk, ones, zeros)
    dis = _tc_dis(degp)
    mp = _tc_prep(x_p, W1, dis, blk)

    for b_prev, w_next in ((b1, W2), (b2, W3), (b3, W4), (b4, W5), (b5, W6)):
        s = _sc_propagate(mp, epk)
        mp = _tc_layer(s, mp, dis, b_prev.reshape(1, -1), w_next, blk)

    s = _sc_propagate(mp, epk)
    return _tc_final(s, mp, dis, b6.reshape(1, -1), batch3d, Wlin,
                     blin.reshape(1, -1), g, blk)
